# baseline (speedup 1.0000x reference)
import functools

import jax
import jax.numpy as jnp
from jax import lax
from jax.experimental import pallas as pl
from jax.experimental.pallas import tpu as pltpu
from jax.experimental.pallas import tpu_sc as plsc

_L = 16


def _make_sc_kernel(B, C, D, V):
    NW = 32
    BPW = B // NW
    CB = 128
    NCH = BPW // CB
    NG = CB // _L
    IW = 80
    NJ = (CB * C) // IW
    KD = D // _L

    mesh = plsc.VectorSubcoreMesh(core_axis_name="c", subcore_axis_name="s")

    @functools.partial(
        pl.kernel,
        mesh=mesh,
        compiler_params=pltpu.CompilerParams(
            needs_layout_passes=False, use_tc_tiling_on_sc=False),
        out_type=jax.ShapeDtypeStruct((B * C,), jnp.float32),
        scratch_types=[
            pltpu.VMEM((CB,), jnp.int32),
            pltpu.VMEM((NJ, IW), jnp.int32),
            pltpu.VMEM((CB, D), jnp.float32),
            pltpu.VMEM((NJ, IW, D), jnp.float32),
            pltpu.VMEM((IW * _L,), jnp.float32),
            pltpu.VMEM((CB * C,), jnp.float32),
            pltpu.SemaphoreType.DMA,
            pltpu.SemaphoreType.DMA,
        ],
    )
    def k(tgt_hbm, ctx_hbm, ttab_hbm, ctab_hbm, out_hbm,
          tgt_idx, ce_idx, we_v, ce_v, acc_buf, out_v, sem_t, sem_c):
        wid = lax.axis_index("s") * 2 + lax.axis_index("c")
        lanes = lax.iota(jnp.int32, _L)

        def chunk_body(ch, carry):
            b_base = pl.multiple_of(wid * BPW + ch * CB, CB)
            pltpu.sync_copy(tgt_hbm.at[pl.ds(b_base, CB)], tgt_idx)
            pltpu.sync_copy(
                ctx_hbm.at[pl.ds(pl.multiple_of((b_base * C) // IW, NJ), NJ)],
                ce_idx)
            dwe = pltpu.async_copy(ttab_hbm.at[tgt_idx], we_v, sem_t)
            dce = [
                pltpu.async_copy(ctab_hbm.at[ce_idx.at[j]], ce_v.at[j], sem_c)
                for j in range(NJ)
            ]
            dwe.wait()
            for d in dce:
                d.wait()

            def g_body(g, gcarry):
                for i in range(_L):
                    b = g * _L + i
                    wv = [we_v[b, pl.ds(kk * _L, _L)] for kk in range(KD)]
                    for c in range(C):
                        q = i * C + c
                        acc = wv[0] * ce_v[g, q, pl.ds(0, _L)]
                        for kk in range(1, KD):
                            acc = acc + wv[kk] * ce_v[g, q, pl.ds(kk * _L, _L)]
                        acc_buf[pl.ds(q * _L, _L)] = acc
                for t in range(C):
                    base_idx = (lanes + t * _L) * _L
                    out_vec = plsc.load_gather(acc_buf, [base_idx])
                    for j in range(1, _L):
                        col = plsc.load_gather(acc_buf, [base_idx + j])
                        out_vec = out_vec + col
                    out_v[pl.ds(g * IW + t * _L, _L)] = out_vec
                return gcarry

            lax.fori_loop(0, NG, g_body, 0)
            pltpu.sync_copy(out_v, out_hbm.at[pl.ds(b_base * C, CB * C)])
            return carry

        lax.fori_loop(0, NCH, chunk_body, 0)

    return k


def kernel(target, context, target_table, context_table):
    B, C = context.shape
    V, D = target_table.shape
    tgt_flat = target.reshape(B)
    ctx_2d = context.reshape((B * C) // 80, 80)
    k = _make_sc_kernel(B, C, D, V)
    out_flat = k(tgt_flat, ctx_2d, target_table, context_table)
    return out_flat.reshape(B, C)
